# baseline (speedup 1.0000x reference)
import functools

import jax
import jax.numpy as jnp
from jax.experimental import pallas as pl
from jax.experimental.pallas import tpu as pltpu

INPUT_SIZE = 128
MEMORY_SIZE = 100000
SPARSITY = 32
VALUE_SIZE = 64
LEARNING_RATE = 0.1
BATCH = 1024

M_PAD = 100352
M_TILE = 1024
NEG = -3.0e38


def _matmul_body(keys_ref, proj_ref, scores_ref):
    j = pl.program_id(0)
    s = jax.lax.dot_general(
        keys_ref[...], proj_ref[...],
        (((1,), (1,)), ((), ())),
        preferred_element_type=jnp.float32,
    )
    col = j * M_TILE + jax.lax.broadcasted_iota(jnp.int32, s.shape, 1)
    scores_ref[...] = jnp.where(col < MEMORY_SIZE, s, NEG)


def _scores(keys, proj_pad):
    grid = (M_PAD // M_TILE,)
    return pl.pallas_call(
        _matmul_body,
        grid=grid,
        in_specs=[
            pl.BlockSpec((BATCH, INPUT_SIZE), lambda j: (0, 0)),
            pl.BlockSpec((M_TILE, INPUT_SIZE), lambda j: (j, 0)),
        ],
        out_specs=pl.BlockSpec((BATCH, M_TILE), lambda j: (0, j)),
        out_shape=jax.ShapeDtypeStruct((BATCH, M_PAD), jnp.float32),
    )(keys, proj_pad)


def kernel(keys, targets, proj, mem_value):
    proj_pad = jnp.pad(proj, ((0, M_PAD - MEMORY_SIZE), (0, 0)))
    scores = _scores(keys, proj_pad)
    _, indices = jax.lax.top_k(scores, SPARSITY)
    retrieved = mem_value[indices].sum(axis=1)
    deltas = (targets - retrieved) / SPARSITY * LEARNING_RATE
    B, S = indices.shape
    V = targets.shape[1]
    flat_idx = indices.reshape(-1)
    flat_delta = jnp.broadcast_to(deltas[:, None, :], (B, S, V)).reshape(-1, V)
    return mem_value.at[flat_idx].add(flat_delta)


# speedup vs baseline: 3.4032x; 3.4032x over previous
import functools

import jax
import jax.numpy as jnp
from jax.experimental import pallas as pl
from jax.experimental.pallas import tpu as pltpu

INPUT_SIZE = 128
MEMORY_SIZE = 100000
SPARSITY = 32
VALUE_SIZE = 64
LEARNING_RATE = 0.1
BATCH = 1024

M_PAD = 100352
M_TILE = 1024
NEG = -3.0e38


def _matmul_body(keys_ref, proj_ref, scores_ref, bmax_ref):
    j = pl.program_id(0)
    s = jax.lax.dot_general(
        keys_ref[...], proj_ref[...],
        (((1,), (1,)), ((), ())),
        preferred_element_type=jnp.float32,
    )
    col = j * M_TILE + jax.lax.broadcasted_iota(jnp.int32, s.shape, 1)
    s = jnp.where(col < MEMORY_SIZE, s, NEG)
    scores_ref[...] = s
    bmax_ref[...] = jnp.max(
        s.reshape(BATCH, M_TILE // 128, 128), axis=2)[None]


def _scores(keys, proj_pad):
    grid = (M_PAD // M_TILE,)
    return pl.pallas_call(
        _matmul_body,
        grid=grid,
        in_specs=[
            pl.BlockSpec((BATCH, INPUT_SIZE), lambda j: (0, 0)),
            pl.BlockSpec((M_TILE, INPUT_SIZE), lambda j: (j, 0)),
        ],
        out_specs=[
            pl.BlockSpec((BATCH, M_TILE), lambda j: (0, j)),
            pl.BlockSpec((1, BATCH, M_TILE // 128), lambda j: (j, 0, 0)),
        ],
        out_shape=[
            jax.ShapeDtypeStruct((BATCH, M_PAD), jnp.float32),
            jax.ShapeDtypeStruct(
                (M_PAD // M_TILE, BATCH, M_TILE // 128), jnp.float32),
        ],
    )(keys, proj_pad)


def kernel(keys, targets, proj, mem_value):
    proj_pad = jnp.pad(proj, ((0, M_PAD - MEMORY_SIZE), (0, 0)))
    scores, bmax3 = _scores(keys, proj_pad)
    bmax = bmax3.transpose(1, 0, 2).reshape(BATCH, M_PAD // 128)
    _, bids = jax.lax.top_k(bmax, SPARSITY)
    cand_cols = (bids[:, :, None] * 128
                 + jnp.arange(128, dtype=jnp.int32)[None, None, :]
                 ).reshape(BATCH, SPARSITY * 128)
    cand = jnp.take_along_axis(scores, cand_cols, axis=1)
    _, cpos = jax.lax.top_k(cand, SPARSITY)
    indices = jnp.take_along_axis(cand_cols, cpos, axis=1)
    retrieved = mem_value[indices].sum(axis=1)
    deltas = (targets - retrieved) / SPARSITY * LEARNING_RATE
    B, S = indices.shape
    V = targets.shape[1]
    flat_idx = indices.reshape(-1)
    flat_delta = jnp.broadcast_to(deltas[:, None, :], (B, S, V)).reshape(-1, V)
    return mem_value.at[flat_idx].add(flat_delta)
